# baseline (speedup 1.0000x reference)
import functools

import jax
import jax.numpy as jnp
from jax import lax
from jax.experimental import pallas as pl
from jax.experimental.pallas import tpu as pltpu
from jax.experimental.pallas import tpu_sc as plsc

N = 2708
E = 10556
M = E + N
NP = 2816
EB = M // 16
NB = NP // 16

_F32 = jnp.float32
_I32 = jnp.int32


_RSC = 4352
_RTC = M - _RSC
_BM = 512
_BC = 4096
_GR = 18
_GC = 4


def _rowsum_body(wl_ref, bl_ref, out_ref):
    j = pl.program_id(1)

    @pl.when(j == 0)
    def _():
        out_ref[...] = bl_ref[...]

    blk = wl_ref[...]

    @pl.when(j < _GC - 1)
    def _():
        out_ref[...] += jnp.sum(blk, axis=1, keepdims=True)

    @pl.when(j == _GC - 1)
    def _():
        cols = j * _BC + lax.broadcasted_iota(_I32, (_BM, _BC), 1)
        out_ref[...] += jnp.sum(jnp.where(cols < M, blk, 0.0), axis=1,
                                keepdims=True)


def _row_sums(Wl, bl):
    out = pl.pallas_call(
        _rowsum_body,
        grid=(_GR, _GC),
        in_specs=[
            pl.BlockSpec((_BM, _BC), lambda i, j: (i, j)),
            pl.BlockSpec((_BM, 1), lambda i, j: (i, 0)),
        ],
        out_specs=pl.BlockSpec((_BM, 1), lambda i, j: (i, 0)),
        out_shape=jax.ShapeDtypeStruct((_GR * _BM, 1), _F32),
        compiler_params=pltpu.CompilerParams(
            dimension_semantics=("parallel", "arbitrary")),
    )(Wl, bl.reshape(M, 1))
    return out.reshape(-1)[:_RTC]


_XB = 1408


def _xw_body(w1_ref, x_ref, b1_ref, out_ref):
    out_ref[...] = lax.dot_general(
        w1_ref[...], x_ref[...],
        dimension_numbers=(((0,), (1,)), ((), ())),
        preferred_element_type=_F32) + b1_ref[...]


def _input_matmul(x, W1, b1):
    return pl.pallas_call(
        _xw_body,
        grid=(2,),
        in_specs=[
            pl.BlockSpec((1433, 32), lambda i: (0, 0)),
            pl.BlockSpec((_XB, 1433), lambda i: (i, 0)),
            pl.BlockSpec((32, 1), lambda i: (0, 0)),
        ],
        out_specs=pl.BlockSpec((32, _XB), lambda i: (0, i)),
        out_shape=jax.ShapeDtypeStruct((32, NP), _F32),
    )(W1, x, b1.reshape(32, 1))


_MESH = plsc.VectorSubcoreMesh(core_axis_name="c", subcore_axis_name="s")
_SC_PARAMS = pltpu.CompilerParams(needs_layout_passes=False)


def _worker_id():
    return lax.axis_index("s") * 2 + lax.axis_index("c")


def _zero_vec(ref, nb=NB):
    def f(i, _):
        ref[pl.ds(i * 16, 16)] = jnp.zeros((16,), _F32)
        return 0
    lax.fori_loop(0, nb, f, 0)


def _copy_vec(src, dst, nb=NB):
    def f(i, _):
        dst[pl.ds(i * 16, 16)] = src[pl.ds(i * 16, 16)]
        return 0
    lax.fori_loop(0, nb, f, 0)


def _relu_vec(ref, nb=NB):
    def f(i, _):
        ref[pl.ds(i * 16, 16)] = jnp.maximum(ref[pl.ds(i * 16, 16)], 0.0)
        return 0
    lax.fori_loop(0, nb, f, 0)


def _scatter_batch(row_v, col_v, ew_v, src, acc, e):
    b = pl.ds(e * 16, 16)
    r = row_v[b]
    c = col_v[b]
    wv = ew_v[b]
    g = plsc.load_gather(src, [r])
    plsc.addupdate_scatter(acc, [c], wv * g)


def _edge_scatter(row_v, col_v, ew_v, src, acc, lo=0, n=EB):
    def f(i, _):
        _scatter_batch(row_v, col_v, ew_v, src, acc, lo + i)
        return 0
    lax.fori_loop(0, n, f, 0)


def _splat(ref, idx):
    return plsc.load_gather(ref, [jnp.full((16,), idx, _I32)])


def _matvec_rows(w_v, b_v, j, f_in, f_out, hprev_hbm, kbuf, hpre):
    bj = _splat(b_v, j)
    first = True
    for k0 in range(0, f_in, 16):
        pltpu.sync_copy(hprev_hbm.at[pl.ds(k0, 16)], kbuf)
        wsp = [_splat(w_v, (k0 + k) * f_out + j) for k in range(16)]
        if first:
            def f(i, _):
                b = pl.ds(i * 16, 16)
                acc16 = bj
                for k in range(16):
                    acc16 = acc16 + wsp[k] * kbuf[k, b]
                hpre[b] = acc16
                return 0
        else:
            def f(i, _):
                b = pl.ds(i * 16, 16)
                acc16 = hpre[b]
                for k in range(16):
                    acc16 = acc16 + wsp[k] * kbuf[k, b]
                hpre[b] = acc16
                return 0
        lax.fori_loop(0, NB, f, 0)
        first = False


def _load_edges(row_h, col_h, ew_h, row_v, col_v, ew_v):
    pltpu.sync_copy(row_h, row_v)
    pltpu.sync_copy(col_h, col_v)
    pltpu.sync_copy(ew_h, ew_v)


_KROWS = _RSC // 32
_NBLK = _KROWS // 4
_RVREG = M // 16


@functools.partial(
    pl.kernel,
    out_type=jax.ShapeDtypeStruct((_RSC,), _F32),
    mesh=_MESH,
    compiler_params=_SC_PARAMS,
    scratch_types=[
        pltpu.VMEM((2, 4, M), _F32),
        pltpu.VMEM((_KROWS,), _F32),
        pltpu.VMEM((_KROWS,), _F32),
        pltpu.SemaphoreType.DMA,
        pltpu.SemaphoreType.DMA,
    ],
)
def _rowsum_sc(wl_h, bl_h, out_h, rbuf, out_v, bl_v, sem0, sem1):
    w = _worker_id()
    base = _RTC + w * _KROWS
    sems = (sem0, sem1)

    handles = [None] * _NBLK
    handles[0] = pltpu.async_copy(wl_h.at[pl.ds(base, 4)], rbuf.at[0], sem0)
    for blk in range(_NBLK):
        if blk + 1 < _NBLK:
            handles[blk + 1] = pltpu.async_copy(
                wl_h.at[pl.ds(base + (blk + 1) * 4, 4)],
                rbuf.at[(blk + 1) % 2], sems[(blk + 1) % 2])
        handles[blk].wait()
        slot = blk % 2

        def row_sum(rr, _):
            def chunk(i, a):
                for u in range(8):
                    a = a + rbuf[slot, rr, pl.ds((i * 8 + u) * 16, 16)]
                return a
            a = lax.fori_loop(0, _RVREG // 8, chunk,
                              jnp.zeros((16,), _F32))
            for t in range(_RVREG - _RVREG % 8, _RVREG):
                a = a + rbuf[slot, rr, pl.ds(t * 16, 16)]
            s = jnp.sum(a)
            plsc.store_scatter(out_v, [jnp.full((16,), blk * 4 + rr, _I32)],
                               jnp.full((16,), s, _F32))
            return 0
        lax.fori_loop(0, 4, row_sum, 0)

    pltpu.sync_copy(bl_h.at[pl.ds(base, _KROWS)], bl_v)

    def addb(i, _):
        b = pl.ds(i * 16, 16)
        out_v[b] += bl_v[b]
        return 0
    lax.fori_loop(0, _KROWS // 16, addb, 0)
    pltpu.sync_copy(out_v, out_h.at[pl.ds(w * _KROWS, _KROWS)])


_EH0 = (EB + 1) // 2


def _compute_ew(ew0_v, row_v, col_v, deg_v, ew_v):
    _zero_vec(deg_v)

    def dscat(e, _):
        b = pl.ds(e * 16, 16)
        plsc.addupdate_scatter(deg_v, [col_v[b]], ew0_v[b])
        return 0
    lax.fori_loop(0, EB, dscat, 0)

    inf = _F32(jnp.inf)

    def ewc(e, _):
        b = pl.ds(e * 16, 16)
        d = plsc.load_gather(deg_v, [row_v[b]])
        d = jnp.where(jnp.abs(d) == inf, _F32(0.0), d)
        ew_v[b] = d * ew0_v[b]
        return 0
    lax.fori_loop(0, EB, ewc, 0)


@functools.partial(
    pl.kernel,
    out_type=(jax.ShapeDtypeStruct((8, NP), _F32),
              jax.ShapeDtypeStruct((8, NP), _F32)),
    mesh=_MESH,
    compiler_params=_SC_PARAMS,
    scratch_types=[
        pltpu.VMEM((M,), _F32),
        pltpu.VMEM((M,), _I32),
        pltpu.VMEM((M,), _I32),
        pltpu.VMEM((M,), _F32),
        pltpu.VMEM((NP,), _F32),
        pltpu.VMEM((16, NP), _F32),
        pltpu.VMEM((NP,), _F32),
        pltpu.VMEM((NP,), _F32),
        pltpu.VMEM((32 * 16,), _F32),
        pltpu.VMEM((16,), _F32),
        pltpu.VMEM((NP,), _I32),
        pltpu.VMEM((NP,), _F32),
        pltpu.VMEM_SHARED((M,), _F32),
        pltpu.VMEM_SHARED((32, NP), _F32),
        pltpu.VMEM_SHARED((16, NP), _F32),
        pltpu.VMEM_SHARED((16, NP), _F32),
        pltpu.VMEM_SHARED((2, 16, NP), _F32),
    ],
)
def _graph_sc(ew0_h, x1_h, w2_h, b2_h, w3_h, b3_h, w4_h, b4_h,
              row_h, col_h, y_h, mask_h, outh_h, outl_h,
              ew0_v, row_v, col_v, ew_v, deg_v, kbuf, hpre, acc,
              w_v, b_v, y_v, m_v, ew_sh, h1_sh, h2_sh, h3_sh, lpa_sh):
    c = lax.axis_index("c")
    s = lax.axis_index("s")

    pltpu.sync_copy(row_h, row_v)
    pltpu.sync_copy(col_h, col_v)

    @pl.when(s == 0)
    def _():
        pltpu.sync_copy(ew0_h, ew0_v)
        _compute_ew(ew0_v, row_v, col_v, deg_v, ew_v)
        pltpu.sync_copy(ew_v, ew_sh)
    plsc.subcore_barrier()

    @pl.when(s != 0)
    def _():
        pltpu.sync_copy(ew_sh, ew_v)

    @pl.when(c == 0)
    def _():
        for ff in range(2):
            f = s * 2 + ff
            pltpu.sync_copy(x1_h.at[f], hpre)
            _zero_vec(acc)
            _edge_scatter(row_v, col_v, ew_v, hpre, acc)
            _relu_vec(acc)
            pltpu.sync_copy(acc, h1_sh.at[f])
        plsc.subcore_barrier()

        pltpu.sync_copy(w2_h, w_v.at[pl.ds(0, 32 * 16)])
        pltpu.sync_copy(b2_h, b_v)
        _matvec_rows(w_v, b_v, s, 32, 16, h1_sh, kbuf, hpre)
        _zero_vec(acc)
        _edge_scatter(row_v, col_v, ew_v, hpre, acc)
        _relu_vec(acc)
        pltpu.sync_copy(acc, h2_sh.at[s])
        plsc.subcore_barrier()

        pltpu.sync_copy(w3_h, w_v.at[pl.ds(0, 16 * 16)])
        pltpu.sync_copy(b3_h, b_v)
        _matvec_rows(w_v, b_v, s, 16, 16, h2_sh, kbuf, hpre)
        _zero_vec(acc)
        _edge_scatter(row_v, col_v, ew_v, hpre, acc)
        _relu_vec(acc)
        pltpu.sync_copy(acc, h3_sh.at[s])
        plsc.subcore_barrier()

        @pl.when(s < 7)
        def _():
            pltpu.sync_copy(w4_h, w_v.at[pl.ds(0, 16 * 8)])
            pltpu.sync_copy(b4_h, b_v)
            _matvec_rows(w_v, b_v, s, 16, 8, h3_sh, kbuf, hpre)
            _zero_vec(acc)
            _edge_scatter(row_v, col_v, ew_v, hpre, acc)
            pltpu.sync_copy(acc, outh_h.at[s])

    @pl.when(c == 1)
    def _():
        active = s < 14
        cls = lax.rem(s, 7)
        half = s // 7

        @pl.when(active)
        def _():
            pltpu.sync_copy(y_h, y_v)
            pltpu.sync_copy(mask_h, m_v)

            def build(i, _):
                b = pl.ds(i * 16, 16)
                hpre[b] = jnp.where(y_v[b] == cls, m_v[b], _F32(0.0))
                return 0
            lax.fori_loop(0, NB, build, 0)

        lo = half * _EH0
        n = jnp.where(half == 0, _EH0, EB - _EH0)
        partner = jnp.where(half == 0, s + 7, s - 7)

        for k in range(5):
            @pl.when(active)
            def _():
                _zero_vec(acc)
                _edge_scatter(row_v, col_v, ew_v, hpre, acc, lo=lo, n=n)
                pltpu.sync_copy(acc, lpa_sh.at[k % 2, s])
            plsc.subcore_barrier()

            @pl.when(active)
            def _():
                pltpu.sync_copy(lpa_sh.at[k % 2, partner], deg_v)

                def addf(i, _):
                    b = pl.ds(i * 16, 16)
                    hpre[b] = acc[b] + deg_v[b]
                    return 0
                lax.fori_loop(0, NB, addf, 0)

        @pl.when(active & (half == 0))
        def _():
            pltpu.sync_copy(hpre, outl_h.at[cls])


def _lsm_body(h_ref, l_ref, oh_ref, ol_ref):
    for src, dst in ((h_ref, oh_ref), (l_ref, ol_ref)):
        v = src[0:7, :]
        m = jnp.max(v, axis=0, keepdims=True)
        e = jnp.exp(v - m)
        s = jnp.sum(e, axis=0, keepdims=True)
        dst[...] = v - m - jnp.log(s)


def _log_softmax(h4_t, lab_t):
    return pl.pallas_call(
        _lsm_body,
        out_shape=(jax.ShapeDtypeStruct((7, NP), _F32),
                   jax.ShapeDtypeStruct((7, NP), _F32)),
    )(h4_t, lab_t)


def kernel(x, edge_index, y, train_mask, W1, b1, W2, b2, W3, b3, W4, b4,
           Wl, bl):
    loops = jnp.arange(N, dtype=_I32)
    row = jnp.concatenate([edge_index[0].astype(_I32), loops])
    col = jnp.concatenate([edge_index[1].astype(_I32), loops])

    ew0_sc = _rowsum_sc(Wl, bl)
    ew0 = jnp.concatenate([_row_sums(Wl, bl), ew0_sc])

    x1pre_t = _input_matmul(x, W1, b1)
    W4p = jnp.pad(W4, ((0, 0), (0, 1))).reshape(-1)
    b4p = jnp.pad(b4, (0, 9))
    yp = jnp.pad(y.astype(_I32), (0, NP - N))
    mp = jnp.pad(train_mask.astype(_F32), (0, NP - N))
    h4, lab = _graph_sc(ew0, x1pre_t, W2.reshape(-1), b2, W3.reshape(-1), b3,
                        W4p, b4p, row, col, yp, mp)

    oh, ol = _log_softmax(h4, lab)
    return oh[:, :N].T, ol[:, :N].T


# speedup vs baseline: 1.0091x; 1.0091x over previous
import functools

import jax
import jax.numpy as jnp
from jax import lax
from jax.experimental import pallas as pl
from jax.experimental.pallas import tpu as pltpu
from jax.experimental.pallas import tpu_sc as plsc

N = 2708
E = 10556
M = E + N
NP = 2816
EB = M // 16
NB = NP // 16

_F32 = jnp.float32
_I32 = jnp.int32


_RSC = 4096
_RTC = M - _RSC
_BM = 512
_BC = 4096
_GR = 18
_GC = 4


def _rowsum_body(wl_ref, bl_ref, out_ref):
    j = pl.program_id(1)

    @pl.when(j == 0)
    def _():
        out_ref[...] = bl_ref[...]

    blk = wl_ref[...]

    @pl.when(j < _GC - 1)
    def _():
        out_ref[...] += jnp.sum(blk, axis=1, keepdims=True)

    @pl.when(j == _GC - 1)
    def _():
        cols = j * _BC + lax.broadcasted_iota(_I32, (_BM, _BC), 1)
        out_ref[...] += jnp.sum(jnp.where(cols < M, blk, 0.0), axis=1,
                                keepdims=True)


def _row_sums(Wl, bl):
    out = pl.pallas_call(
        _rowsum_body,
        grid=(_GR, _GC),
        in_specs=[
            pl.BlockSpec((_BM, _BC), lambda i, j: (i, j)),
            pl.BlockSpec((_BM, 1), lambda i, j: (i, 0)),
        ],
        out_specs=pl.BlockSpec((_BM, 1), lambda i, j: (i, 0)),
        out_shape=jax.ShapeDtypeStruct((_GR * _BM, 1), _F32),
        compiler_params=pltpu.CompilerParams(
            dimension_semantics=("parallel", "arbitrary")),
    )(Wl, bl.reshape(M, 1))
    return out.reshape(-1)[:_RTC]


_XB = 1408


def _xw_body(w1_ref, x_ref, b1_ref, out_ref):
    out_ref[...] = lax.dot_general(
        w1_ref[...], x_ref[...],
        dimension_numbers=(((0,), (1,)), ((), ())),
        preferred_element_type=_F32) + b1_ref[...]


def _input_matmul(x, W1, b1):
    return pl.pallas_call(
        _xw_body,
        grid=(2,),
        in_specs=[
            pl.BlockSpec((1433, 32), lambda i: (0, 0)),
            pl.BlockSpec((_XB, 1433), lambda i: (i, 0)),
            pl.BlockSpec((32, 1), lambda i: (0, 0)),
        ],
        out_specs=pl.BlockSpec((32, _XB), lambda i: (0, i)),
        out_shape=jax.ShapeDtypeStruct((32, NP), _F32),
    )(W1, x, b1.reshape(32, 1))


_MESH = plsc.VectorSubcoreMesh(core_axis_name="c", subcore_axis_name="s")
_SC_PARAMS = pltpu.CompilerParams(needs_layout_passes=False)


def _worker_id():
    return lax.axis_index("s") * 2 + lax.axis_index("c")


def _zero_vec(ref, nb=NB):
    def f(i, _):
        ref[pl.ds(i * 16, 16)] = jnp.zeros((16,), _F32)
        return 0
    lax.fori_loop(0, nb, f, 0)


def _copy_vec(src, dst, nb=NB):
    def f(i, _):
        dst[pl.ds(i * 16, 16)] = src[pl.ds(i * 16, 16)]
        return 0
    lax.fori_loop(0, nb, f, 0)


def _relu_vec(ref, nb=NB):
    def f(i, _):
        ref[pl.ds(i * 16, 16)] = jnp.maximum(ref[pl.ds(i * 16, 16)], 0.0)
        return 0
    lax.fori_loop(0, nb, f, 0)


def _scatter_batch(row_v, col_v, ew_v, src, acc, e):
    b = pl.ds(e * 16, 16)
    r = row_v[b]
    c = col_v[b]
    wv = ew_v[b]
    g = plsc.load_gather(src, [r])
    plsc.addupdate_scatter(acc, [c], wv * g)


def _edge_scatter(row_v, col_v, ew_v, src, acc, lo=0, n=EB):
    def f(i, _):
        _scatter_batch(row_v, col_v, ew_v, src, acc, lo + i)
        return 0
    lax.fori_loop(0, n, f, 0)


def _splat(ref, idx):
    return plsc.load_gather(ref, [jnp.full((16,), idx, _I32)])


def _matvec_rows(w_v, b_v, j, f_in, f_out, hprev_hbm, kbuf, hpre):
    bj = _splat(b_v, j)
    first = True
    for k0 in range(0, f_in, 16):
        pltpu.sync_copy(hprev_hbm.at[pl.ds(k0, 16)], kbuf)
        wsp = [_splat(w_v, (k0 + k) * f_out + j) for k in range(16)]
        if first:
            def f(i, _):
                b = pl.ds(i * 16, 16)
                acc16 = bj
                for k in range(16):
                    acc16 = acc16 + wsp[k] * kbuf[k, b]
                hpre[b] = acc16
                return 0
        else:
            def f(i, _):
                b = pl.ds(i * 16, 16)
                acc16 = hpre[b]
                for k in range(16):
                    acc16 = acc16 + wsp[k] * kbuf[k, b]
                hpre[b] = acc16
                return 0
        lax.fori_loop(0, NB, f, 0)
        first = False


def _load_edges(row_h, col_h, ew_h, row_v, col_v, ew_v):
    pltpu.sync_copy(row_h, row_v)
    pltpu.sync_copy(col_h, col_v)
    pltpu.sync_copy(ew_h, ew_v)


_KROWS = _RSC // 32
_NBLK = _KROWS // 4
_RVREG = M // 16


@functools.partial(
    pl.kernel,
    out_type=jax.ShapeDtypeStruct((_RSC,), _F32),
    mesh=_MESH,
    compiler_params=_SC_PARAMS,
    scratch_types=[
        pltpu.VMEM((2, 4, M), _F32),
        pltpu.VMEM((_KROWS,), _F32),
        pltpu.VMEM((_KROWS,), _F32),
        pltpu.SemaphoreType.DMA,
        pltpu.SemaphoreType.DMA,
    ],
)
def _rowsum_sc(wl_h, bl_h, out_h, rbuf, out_v, bl_v, sem0, sem1):
    w = _worker_id()
    base = _RTC + w * _KROWS
    sems = (sem0, sem1)

    handles = [None] * _NBLK
    handles[0] = pltpu.async_copy(wl_h.at[pl.ds(base, 4)], rbuf.at[0], sem0)
    for blk in range(_NBLK):
        if blk + 1 < _NBLK:
            handles[blk + 1] = pltpu.async_copy(
                wl_h.at[pl.ds(base + (blk + 1) * 4, 4)],
                rbuf.at[(blk + 1) % 2], sems[(blk + 1) % 2])
        handles[blk].wait()
        slot = blk % 2

        def row_sum(rr, _):
            def chunk(i, a):
                for u in range(8):
                    a = a + rbuf[slot, rr, pl.ds((i * 8 + u) * 16, 16)]
                return a
            a = lax.fori_loop(0, _RVREG // 8, chunk,
                              jnp.zeros((16,), _F32))
            for t in range(_RVREG - _RVREG % 8, _RVREG):
                a = a + rbuf[slot, rr, pl.ds(t * 16, 16)]
            s = jnp.sum(a)
            plsc.store_scatter(out_v, [jnp.full((16,), blk * 4 + rr, _I32)],
                               jnp.full((16,), s, _F32))
            return 0
        lax.fori_loop(0, 4, row_sum, 0)

    pltpu.sync_copy(bl_h.at[pl.ds(base, _KROWS)], bl_v)

    def addb(i, _):
        b = pl.ds(i * 16, 16)
        out_v[b] += bl_v[b]
        return 0
    lax.fori_loop(0, _KROWS // 16, addb, 0)
    pltpu.sync_copy(out_v, out_h.at[pl.ds(w * _KROWS, _KROWS)])


_EH0 = (EB + 1) // 2


def _compute_ew(ew0_v, row_v, col_v, deg_v, ew_v):
    _zero_vec(deg_v)

    def dscat(e, _):
        b = pl.ds(e * 16, 16)
        plsc.addupdate_scatter(deg_v, [col_v[b]], ew0_v[b])
        return 0
    lax.fori_loop(0, EB, dscat, 0)

    inf = _F32(jnp.inf)

    def ewc(e, _):
        b = pl.ds(e * 16, 16)
        d = plsc.load_gather(deg_v, [row_v[b]])
        d = jnp.where(jnp.abs(d) == inf, _F32(0.0), d)
        ew_v[b] = d * ew0_v[b]
        return 0
    lax.fori_loop(0, EB, ewc, 0)


@functools.partial(
    pl.kernel,
    out_type=(jax.ShapeDtypeStruct((8, NP), _F32),
              jax.ShapeDtypeStruct((8, NP), _F32)),
    mesh=_MESH,
    compiler_params=_SC_PARAMS,
    scratch_types=[
        pltpu.VMEM((M,), _F32),
        pltpu.VMEM((M,), _I32),
        pltpu.VMEM((M,), _I32),
        pltpu.VMEM((M,), _F32),
        pltpu.VMEM((NP,), _F32),
        pltpu.VMEM((16, NP), _F32),
        pltpu.VMEM((NP,), _F32),
        pltpu.VMEM((NP,), _F32),
        pltpu.VMEM((32 * 16,), _F32),
        pltpu.VMEM((16,), _F32),
        pltpu.VMEM((NP,), _I32),
        pltpu.VMEM((NP,), _F32),
        pltpu.VMEM_SHARED((M,), _F32),
        pltpu.VMEM_SHARED((32, NP), _F32),
        pltpu.VMEM_SHARED((16, NP), _F32),
        pltpu.VMEM_SHARED((16, NP), _F32),
        pltpu.VMEM_SHARED((2, 16, NP), _F32),
    ],
)
def _graph_sc(ew0_h, x1_h, w2_h, b2_h, w3_h, b3_h, w4_h, b4_h,
              row_h, col_h, y_h, mask_h, outh_h, outl_h,
              ew0_v, row_v, col_v, ew_v, deg_v, kbuf, hpre, acc,
              w_v, b_v, y_v, m_v, ew_sh, h1_sh, h2_sh, h3_sh, lpa_sh):
    c = lax.axis_index("c")
    s = lax.axis_index("s")

    pltpu.sync_copy(row_h, row_v)
    pltpu.sync_copy(col_h, col_v)

    @pl.when(s == 0)
    def _():
        pltpu.sync_copy(ew0_h, ew0_v)
        _compute_ew(ew0_v, row_v, col_v, deg_v, ew_v)
        pltpu.sync_copy(ew_v, ew_sh)
    plsc.subcore_barrier()

    @pl.when(s != 0)
    def _():
        pltpu.sync_copy(ew_sh, ew_v)

    @pl.when(c == 0)
    def _():
        for ff in range(2):
            f = s * 2 + ff
            pltpu.sync_copy(x1_h.at[f], hpre)
            _zero_vec(acc)
            _edge_scatter(row_v, col_v, ew_v, hpre, acc)
            _relu_vec(acc)
            pltpu.sync_copy(acc, h1_sh.at[f])
        plsc.subcore_barrier()

        pltpu.sync_copy(w2_h, w_v.at[pl.ds(0, 32 * 16)])
        pltpu.sync_copy(b2_h, b_v)
        _matvec_rows(w_v, b_v, s, 32, 16, h1_sh, kbuf, hpre)
        _zero_vec(acc)
        _edge_scatter(row_v, col_v, ew_v, hpre, acc)
        _relu_vec(acc)
        pltpu.sync_copy(acc, h2_sh.at[s])
        plsc.subcore_barrier()

        pltpu.sync_copy(w3_h, w_v.at[pl.ds(0, 16 * 16)])
        pltpu.sync_copy(b3_h, b_v)
        _matvec_rows(w_v, b_v, s, 16, 16, h2_sh, kbuf, hpre)
        _zero_vec(acc)
        _edge_scatter(row_v, col_v, ew_v, hpre, acc)
        _relu_vec(acc)
        pltpu.sync_copy(acc, h3_sh.at[s])
        plsc.subcore_barrier()

        @pl.when(s < 7)
        def _():
            pltpu.sync_copy(w4_h, w_v.at[pl.ds(0, 16 * 8)])
            pltpu.sync_copy(b4_h, b_v)
            _matvec_rows(w_v, b_v, s, 16, 8, h3_sh, kbuf, hpre)
            _zero_vec(acc)
            _edge_scatter(row_v, col_v, ew_v, hpre, acc)
            pltpu.sync_copy(acc, outh_h.at[s])

    @pl.when(c == 1)
    def _():
        active = s < 14
        cls = lax.rem(s, 7)
        half = s // 7

        @pl.when(active)
        def _():
            pltpu.sync_copy(y_h, y_v)
            pltpu.sync_copy(mask_h, m_v)

            def build(i, _):
                b = pl.ds(i * 16, 16)
                hpre[b] = jnp.where(y_v[b] == cls, m_v[b], _F32(0.0))
                return 0
            lax.fori_loop(0, NB, build, 0)

        lo = half * _EH0
        n = jnp.where(half == 0, _EH0, EB - _EH0)
        partner = jnp.where(half == 0, s + 7, s - 7)

        for k in range(5):
            @pl.when(active)
            def _():
                _zero_vec(acc)
                _edge_scatter(row_v, col_v, ew_v, hpre, acc, lo=lo, n=n)
                pltpu.sync_copy(acc, lpa_sh.at[k % 2, s])
            plsc.subcore_barrier()

            @pl.when(active)
            def _():
                pltpu.sync_copy(lpa_sh.at[k % 2, partner], deg_v)

                def addf(i, _):
                    b = pl.ds(i * 16, 16)
                    hpre[b] = acc[b] + deg_v[b]
                    return 0
                lax.fori_loop(0, NB, addf, 0)

        @pl.when(active & (half == 0))
        def _():
            pltpu.sync_copy(hpre, outl_h.at[cls])


def _lsm_body(h_ref, l_ref, oh_ref, ol_ref):
    for src, dst in ((h_ref, oh_ref), (l_ref, ol_ref)):
        v = src[0:7, :]
        m = jnp.max(v, axis=0, keepdims=True)
        e = jnp.exp(v - m)
        s = jnp.sum(e, axis=0, keepdims=True)
        dst[...] = v - m - jnp.log(s)


def _log_softmax(h4_t, lab_t):
    return pl.pallas_call(
        _lsm_body,
        out_shape=(jax.ShapeDtypeStruct((7, NP), _F32),
                   jax.ShapeDtypeStruct((7, NP), _F32)),
    )(h4_t, lab_t)


def kernel(x, edge_index, y, train_mask, W1, b1, W2, b2, W3, b3, W4, b4,
           Wl, bl):
    loops = jnp.arange(N, dtype=_I32)
    row = jnp.concatenate([edge_index[0].astype(_I32), loops])
    col = jnp.concatenate([edge_index[1].astype(_I32), loops])

    ew0_sc = _rowsum_sc(Wl, bl)
    ew0 = jnp.concatenate([_row_sums(Wl, bl), ew0_sc])

    x1pre_t = _input_matmul(x, W1, b1)
    W4p = jnp.pad(W4, ((0, 0), (0, 1))).reshape(-1)
    b4p = jnp.pad(b4, (0, 9))
    yp = jnp.pad(y.astype(_I32), (0, NP - N))
    mp = jnp.pad(train_mask.astype(_F32), (0, NP - N))
    h4, lab = _graph_sc(ew0, x1pre_t, W2.reshape(-1), b2, W3.reshape(-1), b3,
                        W4p, b4p, row, col, yp, mp)

    oh, ol = _log_softmax(h4, lab)
    return oh[:, :N].T, ol[:, :N].T


# speedup vs baseline: 1.0128x; 1.0037x over previous
import functools

import jax
import jax.numpy as jnp
from jax import lax
from jax.experimental import pallas as pl
from jax.experimental.pallas import tpu as pltpu
from jax.experimental.pallas import tpu_sc as plsc

N = 2708
E = 10556
M = E + N
NP = 2816
EB = M // 16
NB = NP // 16

_F32 = jnp.float32
_I32 = jnp.int32


_RSC = 4096
_RTC = M - _RSC
_BM = 1024
_BC = 4096
_GR = 9
_GC = 4


def _rowsum_body(wl_ref, bl_ref, out_ref):
    j = pl.program_id(1)

    @pl.when(j == 0)
    def _():
        out_ref[...] = bl_ref[...]

    blk = wl_ref[...]

    @pl.when(j < _GC - 1)
    def _():
        out_ref[...] += jnp.sum(blk, axis=1, keepdims=True)

    @pl.when(j == _GC - 1)
    def _():
        cols = j * _BC + lax.broadcasted_iota(_I32, (_BM, _BC), 1)
        out_ref[...] += jnp.sum(jnp.where(cols < M, blk, 0.0), axis=1,
                                keepdims=True)


def _row_sums(Wl, bl):
    out = pl.pallas_call(
        _rowsum_body,
        grid=(_GR, _GC),
        in_specs=[
            pl.BlockSpec((_BM, _BC), lambda i, j: (i, j)),
            pl.BlockSpec((_BM, 1), lambda i, j: (i, 0)),
        ],
        out_specs=pl.BlockSpec((_BM, 1), lambda i, j: (i, 0)),
        out_shape=jax.ShapeDtypeStruct((_GR * _BM, 1), _F32),
        compiler_params=pltpu.CompilerParams(
            dimension_semantics=("parallel", "arbitrary")),
    )(Wl, bl.reshape(M, 1))
    return out.reshape(-1)[:_RTC]


_XB = 1408


def _xw_body(w1_ref, x_ref, b1_ref, out_ref):
    out_ref[...] = lax.dot_general(
        w1_ref[...], x_ref[...],
        dimension_numbers=(((0,), (1,)), ((), ())),
        preferred_element_type=_F32) + b1_ref[...]


def _input_matmul(x, W1, b1):
    return pl.pallas_call(
        _xw_body,
        grid=(2,),
        in_specs=[
            pl.BlockSpec((1433, 32), lambda i: (0, 0)),
            pl.BlockSpec((_XB, 1433), lambda i: (i, 0)),
            pl.BlockSpec((32, 1), lambda i: (0, 0)),
        ],
        out_specs=pl.BlockSpec((32, _XB), lambda i: (0, i)),
        out_shape=jax.ShapeDtypeStruct((32, NP), _F32),
    )(W1, x, b1.reshape(32, 1))


_MESH = plsc.VectorSubcoreMesh(core_axis_name="c", subcore_axis_name="s")
_SC_PARAMS = pltpu.CompilerParams(needs_layout_passes=False)


def _worker_id():
    return lax.axis_index("s") * 2 + lax.axis_index("c")


def _zero_vec(ref, nb=NB):
    def f(i, _):
        ref[pl.ds(i * 16, 16)] = jnp.zeros((16,), _F32)
        return 0
    lax.fori_loop(0, nb, f, 0)


def _copy_vec(src, dst, nb=NB):
    def f(i, _):
        dst[pl.ds(i * 16, 16)] = src[pl.ds(i * 16, 16)]
        return 0
    lax.fori_loop(0, nb, f, 0)


def _relu_vec(ref, nb=NB):
    def f(i, _):
        ref[pl.ds(i * 16, 16)] = jnp.maximum(ref[pl.ds(i * 16, 16)], 0.0)
        return 0
    lax.fori_loop(0, nb, f, 0)


def _scatter_batch(row_v, col_v, ew_v, src, acc, e):
    b = pl.ds(e * 16, 16)
    r = row_v[b]
    c = col_v[b]
    wv = ew_v[b]
    g = plsc.load_gather(src, [r])
    plsc.addupdate_scatter(acc, [c], wv * g)


def _edge_scatter(row_v, col_v, ew_v, src, acc, lo=0, n=EB):
    def f(i, _):
        _scatter_batch(row_v, col_v, ew_v, src, acc, lo + i)
        return 0
    lax.fori_loop(0, n, f, 0)


def _splat(ref, idx):
    return plsc.load_gather(ref, [jnp.full((16,), idx, _I32)])


def _matvec_rows(w_v, b_v, j, f_in, f_out, hprev_hbm, kbuf, hpre):
    bj = _splat(b_v, j)
    first = True
    for k0 in range(0, f_in, 16):
        pltpu.sync_copy(hprev_hbm.at[pl.ds(k0, 16)], kbuf)
        wsp = [_splat(w_v, (k0 + k) * f_out + j) for k in range(16)]
        if first:
            def f(i, _):
                b = pl.ds(i * 16, 16)
                acc16 = bj
                for k in range(16):
                    acc16 = acc16 + wsp[k] * kbuf[k, b]
                hpre[b] = acc16
                return 0
        else:
            def f(i, _):
                b = pl.ds(i * 16, 16)
                acc16 = hpre[b]
                for k in range(16):
                    acc16 = acc16 + wsp[k] * kbuf[k, b]
                hpre[b] = acc16
                return 0
        lax.fori_loop(0, NB, f, 0)
        first = False


def _load_edges(row_h, col_h, ew_h, row_v, col_v, ew_v):
    pltpu.sync_copy(row_h, row_v)
    pltpu.sync_copy(col_h, col_v)
    pltpu.sync_copy(ew_h, ew_v)


_KROWS = _RSC // 32
_NBLK = _KROWS // 4
_RVREG = M // 16


@functools.partial(
    pl.kernel,
    out_type=jax.ShapeDtypeStruct((_RSC,), _F32),
    mesh=_MESH,
    compiler_params=_SC_PARAMS,
    scratch_types=[
        pltpu.VMEM((2, 4, M), _F32),
        pltpu.VMEM((_KROWS,), _F32),
        pltpu.VMEM((_KROWS,), _F32),
        pltpu.SemaphoreType.DMA,
        pltpu.SemaphoreType.DMA,
    ],
)
def _rowsum_sc(wl_h, bl_h, out_h, rbuf, out_v, bl_v, sem0, sem1):
    w = _worker_id()
    base = _RTC + w * _KROWS
    sems = (sem0, sem1)

    handles = [None] * _NBLK
    handles[0] = pltpu.async_copy(wl_h.at[pl.ds(base, 4)], rbuf.at[0], sem0)
    for blk in range(_NBLK):
        if blk + 1 < _NBLK:
            handles[blk + 1] = pltpu.async_copy(
                wl_h.at[pl.ds(base + (blk + 1) * 4, 4)],
                rbuf.at[(blk + 1) % 2], sems[(blk + 1) % 2])
        handles[blk].wait()
        slot = blk % 2

        def row_sum(rr, _):
            def chunk(i, a):
                for u in range(8):
                    a = a + rbuf[slot, rr, pl.ds((i * 8 + u) * 16, 16)]
                return a
            a = lax.fori_loop(0, _RVREG // 8, chunk,
                              jnp.zeros((16,), _F32))
            for t in range(_RVREG - _RVREG % 8, _RVREG):
                a = a + rbuf[slot, rr, pl.ds(t * 16, 16)]
            s = jnp.sum(a)
            plsc.store_scatter(out_v, [jnp.full((16,), blk * 4 + rr, _I32)],
                               jnp.full((16,), s, _F32))
            return 0
        lax.fori_loop(0, 4, row_sum, 0)

    pltpu.sync_copy(bl_h.at[pl.ds(base, _KROWS)], bl_v)

    def addb(i, _):
        b = pl.ds(i * 16, 16)
        out_v[b] += bl_v[b]
        return 0
    lax.fori_loop(0, _KROWS // 16, addb, 0)
    pltpu.sync_copy(out_v, out_h.at[pl.ds(w * _KROWS, _KROWS)])


_EH0 = (EB + 1) // 2


def _compute_ew(ew0_v, row_v, col_v, deg_v, ew_v):
    _zero_vec(deg_v)

    def dscat(e, _):
        b = pl.ds(e * 16, 16)
        plsc.addupdate_scatter(deg_v, [col_v[b]], ew0_v[b])
        return 0
    lax.fori_loop(0, EB, dscat, 0)

    inf = _F32(jnp.inf)

    def ewc(e, _):
        b = pl.ds(e * 16, 16)
        d = plsc.load_gather(deg_v, [row_v[b]])
        d = jnp.where(jnp.abs(d) == inf, _F32(0.0), d)
        ew_v[b] = d * ew0_v[b]
        return 0
    lax.fori_loop(0, EB, ewc, 0)


@functools.partial(
    pl.kernel,
    out_type=(jax.ShapeDtypeStruct((8, NP), _F32),
              jax.ShapeDtypeStruct((8, NP), _F32)),
    mesh=_MESH,
    compiler_params=_SC_PARAMS,
    scratch_types=[
        pltpu.VMEM((M,), _F32),
        pltpu.VMEM((M,), _I32),
        pltpu.VMEM((M,), _I32),
        pltpu.VMEM((M,), _F32),
        pltpu.VMEM((NP,), _F32),
        pltpu.VMEM((16, NP), _F32),
        pltpu.VMEM((NP,), _F32),
        pltpu.VMEM((NP,), _F32),
        pltpu.VMEM((32 * 16,), _F32),
        pltpu.VMEM((16,), _F32),
        pltpu.VMEM((NP,), _I32),
        pltpu.VMEM((NP,), _F32),
        pltpu.VMEM_SHARED((M,), _F32),
        pltpu.VMEM_SHARED((32, NP), _F32),
        pltpu.VMEM_SHARED((16, NP), _F32),
        pltpu.VMEM_SHARED((16, NP), _F32),
        pltpu.VMEM_SHARED((2, 16, NP), _F32),
    ],
)
def _graph_sc(ew0_h, x1_h, w2_h, b2_h, w3_h, b3_h, w4_h, b4_h,
              row_h, col_h, y_h, mask_h, outh_h, outl_h,
              ew0_v, row_v, col_v, ew_v, deg_v, kbuf, hpre, acc,
              w_v, b_v, y_v, m_v, ew_sh, h1_sh, h2_sh, h3_sh, lpa_sh):
    c = lax.axis_index("c")
    s = lax.axis_index("s")

    pltpu.sync_copy(row_h, row_v)
    pltpu.sync_copy(col_h, col_v)

    @pl.when(s == 0)
    def _():
        pltpu.sync_copy(ew0_h, ew0_v)
        _compute_ew(ew0_v, row_v, col_v, deg_v, ew_v)
        pltpu.sync_copy(ew_v, ew_sh)
    plsc.subcore_barrier()

    @pl.when(s != 0)
    def _():
        pltpu.sync_copy(ew_sh, ew_v)

    @pl.when(c == 0)
    def _():
        for ff in range(2):
            f = s * 2 + ff
            pltpu.sync_copy(x1_h.at[f], hpre)
            _zero_vec(acc)
            _edge_scatter(row_v, col_v, ew_v, hpre, acc)
            _relu_vec(acc)
            pltpu.sync_copy(acc, h1_sh.at[f])
        plsc.subcore_barrier()

        pltpu.sync_copy(w2_h, w_v.at[pl.ds(0, 32 * 16)])
        pltpu.sync_copy(b2_h, b_v)
        _matvec_rows(w_v, b_v, s, 32, 16, h1_sh, kbuf, hpre)
        _zero_vec(acc)
        _edge_scatter(row_v, col_v, ew_v, hpre, acc)
        _relu_vec(acc)
        pltpu.sync_copy(acc, h2_sh.at[s])
        plsc.subcore_barrier()

        pltpu.sync_copy(w3_h, w_v.at[pl.ds(0, 16 * 16)])
        pltpu.sync_copy(b3_h, b_v)
        _matvec_rows(w_v, b_v, s, 16, 16, h2_sh, kbuf, hpre)
        _zero_vec(acc)
        _edge_scatter(row_v, col_v, ew_v, hpre, acc)
        _relu_vec(acc)
        pltpu.sync_copy(acc, h3_sh.at[s])
        plsc.subcore_barrier()

        @pl.when(s < 7)
        def _():
            pltpu.sync_copy(w4_h, w_v.at[pl.ds(0, 16 * 8)])
            pltpu.sync_copy(b4_h, b_v)
            _matvec_rows(w_v, b_v, s, 16, 8, h3_sh, kbuf, hpre)
            _zero_vec(acc)
            _edge_scatter(row_v, col_v, ew_v, hpre, acc)
            pltpu.sync_copy(acc, outh_h.at[s])

    @pl.when(c == 1)
    def _():
        active = s < 14
        cls = lax.rem(s, 7)
        half = s // 7

        @pl.when(active)
        def _():
            pltpu.sync_copy(y_h, y_v)
            pltpu.sync_copy(mask_h, m_v)

            def build(i, _):
                b = pl.ds(i * 16, 16)
                hpre[b] = jnp.where(y_v[b] == cls, m_v[b], _F32(0.0))
                return 0
            lax.fori_loop(0, NB, build, 0)

        lo = half * _EH0
        n = jnp.where(half == 0, _EH0, EB - _EH0)
        partner = jnp.where(half == 0, s + 7, s - 7)

        for k in range(5):
            @pl.when(active)
            def _():
                _zero_vec(acc)
                _edge_scatter(row_v, col_v, ew_v, hpre, acc, lo=lo, n=n)
                pltpu.sync_copy(acc, lpa_sh.at[k % 2, s])
            plsc.subcore_barrier()

            @pl.when(active)
            def _():
                pltpu.sync_copy(lpa_sh.at[k % 2, partner], deg_v)

                def addf(i, _):
                    b = pl.ds(i * 16, 16)
                    hpre[b] = acc[b] + deg_v[b]
                    return 0
                lax.fori_loop(0, NB, addf, 0)

        @pl.when(active & (half == 0))
        def _():
            pltpu.sync_copy(hpre, outl_h.at[cls])


def _lsm_body(h_ref, l_ref, oh_ref, ol_ref):
    for src, dst in ((h_ref, oh_ref), (l_ref, ol_ref)):
        v = src[0:7, :]
        m = jnp.max(v, axis=0, keepdims=True)
        e = jnp.exp(v - m)
        s = jnp.sum(e, axis=0, keepdims=True)
        dst[...] = v - m - jnp.log(s)


def _log_softmax(h4_t, lab_t):
    return pl.pallas_call(
        _lsm_body,
        out_shape=(jax.ShapeDtypeStruct((7, NP), _F32),
                   jax.ShapeDtypeStruct((7, NP), _F32)),
    )(h4_t, lab_t)


def kernel(x, edge_index, y, train_mask, W1, b1, W2, b2, W3, b3, W4, b4,
           Wl, bl):
    loops = jnp.arange(N, dtype=_I32)
    row = jnp.concatenate([edge_index[0].astype(_I32), loops])
    col = jnp.concatenate([edge_index[1].astype(_I32), loops])

    ew0_sc = _rowsum_sc(Wl, bl)
    ew0 = jnp.concatenate([_row_sums(Wl, bl), ew0_sc])

    x1pre_t = _input_matmul(x, W1, b1)
    W4p = jnp.pad(W4, ((0, 0), (0, 1))).reshape(-1)
    b4p = jnp.pad(b4, (0, 9))
    yp = jnp.pad(y.astype(_I32), (0, NP - N))
    mp = jnp.pad(train_mask.astype(_F32), (0, NP - N))
    h4, lab = _graph_sc(ew0, x1pre_t, W2.reshape(-1), b2, W3.reshape(-1), b3,
                        W4p, b4p, row, col, yp, mp)

    oh, ol = _log_softmax(h4, lab)
    return oh[:, :N].T, ol[:, :N].T
